# baseline (speedup 1.0000x reference)
import jax
import jax.numpy as jnp
from jax import lax
from jax.experimental import pallas as pl
from jax.experimental.pallas import tpu as pltpu
from jax.experimental.pallas import tpu_sc as plsc

_B_BLK = 256
_C_BLK = 8


def _vq_tc_kernel(x_ref, cbt_ref, csq_ref, oh_ref, idx_ref):
    num_c = pl.num_programs(1)
    c_pid = pl.program_id(1)
    book_size = cbt_ref.shape[2]
    for c in range(_C_BLK):
        xb = x_ref[c]
        cbt = cbt_ref[c]
        cross2 = jnp.dot(xb, cbt, preferred_element_type=jnp.float32)
        x_sq = jnp.sum(xb * xb, axis=1, keepdims=True)
        dist = (x_sq + cross2) + csq_ref[c]
        idx = jnp.argmin(dist, axis=1)
        k_iota = jax.lax.broadcasted_iota(jnp.int32, dist.shape, 1)
        onehot = (k_iota == idx[:, None]).astype(jnp.float32)
        oh_ref[:, c * book_size:(c + 1) * book_size] = onehot
        c_glob = c_pid * _C_BLK + c
        idx_ref[c] = (idx + c_glob * book_size)[:, None]
    del num_c


def _vq_sc_gather(table_hbm, idx_hbm, out_hbm, idx_v, rows_v, sem):
    wid = lax.axis_index("s") * 2 + lax.axis_index("c")
    n_rows = rows_v.shape[0]
    n_chunks = idx_v.shape[0]
    chunk = idx_v.shape[1]
    base = wid * n_rows
    pltpu.sync_copy(idx_hbm.at[pl.ds(wid * n_chunks, n_chunks)], idx_v)
    copies = []
    for j in range(n_chunks):
        copies.append(pltpu.async_copy(
            table_hbm.at[idx_v.at[j]],
            rows_v.at[pl.ds(j * chunk, chunk)], sem))
    for cp in copies:
        cp.wait()
    pltpu.sync_copy(rows_v, out_hbm.at[pl.ds(base, n_rows)])


def kernel(x, codebook):
    batch, embed = x.shape
    dim_codes, book_size, dim_embedding = codebook.shape
    cbt = codebook.transpose(0, 2, 1) * -2.0
    c_sq = jnp.sum(codebook * codebook, axis=2)[:, None, :]
    xt = x.reshape(batch, dim_codes, dim_embedding).transpose(1, 0, 2)

    grid = (batch // _B_BLK, dim_codes // _C_BLK)
    oh, idx3 = pl.pallas_call(
        _vq_tc_kernel,
        grid=grid,
        in_specs=[
            pl.BlockSpec((_C_BLK, _B_BLK, dim_embedding), lambda b, c: (c, b, 0)),
            pl.BlockSpec((_C_BLK, dim_embedding, book_size), lambda b, c: (c, 0, 0)),
            pl.BlockSpec((_C_BLK, 1, book_size), lambda b, c: (c, 0, 0)),
        ],
        out_specs=[
            pl.BlockSpec((_B_BLK, _C_BLK * book_size), lambda b, c: (b, c)),
            pl.BlockSpec((_C_BLK, _B_BLK, 1), lambda b, c: (c, b, 0)),
        ],
        out_shape=[
            jax.ShapeDtypeStruct((batch, dim_codes * book_size), jnp.float32),
            jax.ShapeDtypeStruct((dim_codes, batch, 1), jnp.int32),
        ],
    )(xt, cbt, c_sq)

    n_rows_total = dim_codes * batch
    n_workers = 32
    rows_per_w = n_rows_total // n_workers
    chunk = 128
    table = codebook.reshape(dim_codes * book_size, dim_embedding)
    idx_flat = idx3.reshape(n_rows_total // chunk, chunk)

    mesh = plsc.VectorSubcoreMesh(core_axis_name="c", subcore_axis_name="s")
    gather_call = pl.kernel(
        _vq_sc_gather,
        mesh=mesh,
        compiler_params=pltpu.CompilerParams(use_tc_tiling_on_sc=False),
        out_type=jax.ShapeDtypeStruct((n_rows_total, dim_embedding), jnp.float32),
        scratch_types=[
            pltpu.VMEM((rows_per_w // chunk, chunk), jnp.int32),
            pltpu.VMEM((rows_per_w, dim_embedding), jnp.float32),
            pltpu.SemaphoreType.DMA,
        ],
    )
    ce_rows = gather_call(table, idx_flat)

    ce = (ce_rows.reshape(dim_codes, batch, dim_embedding)
          .transpose(1, 0, 2).reshape(batch, embed))
    return (ce, ce, oh.reshape(batch, dim_codes, book_size))


# speedup vs baseline: 1.0445x; 1.0445x over previous
import jax
import jax.numpy as jnp
from jax import lax
from jax.experimental import pallas as pl
from jax.experimental.pallas import tpu as pltpu
from jax.experimental.pallas import tpu_sc as plsc

_B_BLK = 256
_C_BLK = 8


def _vq_tc_kernel(x_ref, cbt_ref, csq_ref, oh_ref, idx_ref):
    num_c = pl.num_programs(1)
    c_pid = pl.program_id(1)
    book_size = cbt_ref.shape[2]
    for c in range(_C_BLK):
        xb = x_ref[c]
        cbt = cbt_ref[c]
        cross2 = jnp.dot(xb, cbt, preferred_element_type=jnp.float32)
        x_sq = jnp.sum(xb * xb, axis=1, keepdims=True)
        dist = (x_sq + cross2) + csq_ref[c]
        idx = jnp.argmin(dist, axis=1)
        k_iota = jax.lax.broadcasted_iota(jnp.int32, dist.shape, 1)
        onehot = (k_iota == idx[:, None]).astype(jnp.float32)
        oh_ref[:, c * book_size:(c + 1) * book_size] = onehot
        c_glob = c_pid * _C_BLK + c
        idx_ref[c] = (idx + c_glob * book_size)[:, None]
    del num_c


def _vq_sc_gather(table_hbm, idx_hbm, dst_hbm, out_hbm, idx_v, dst_v, rows_v,
                  sem, sem2):
    wid = lax.axis_index("s") * 2 + lax.axis_index("c")
    n_chunks = idx_v.shape[0]
    chunk = idx_v.shape[1]
    pltpu.sync_copy(idx_hbm.at[pl.ds(wid * n_chunks, n_chunks)], idx_v)
    pltpu.sync_copy(dst_hbm.at[pl.ds(wid * n_chunks, n_chunks)], dst_v)
    gathers = []
    for j in range(n_chunks):
        gathers.append(pltpu.async_copy(
            table_hbm.at[idx_v.at[j]],
            rows_v.at[pl.ds(j * chunk, chunk)], sem))
    for cp in gathers:
        cp.wait()
    scatters = []
    for j in range(n_chunks):
        scatters.append(pltpu.async_copy(
            rows_v.at[pl.ds(j * chunk, chunk)],
            out_hbm.at[dst_v.at[j]], sem2))
    for cp in scatters:
        cp.wait()


def kernel(x, codebook):
    batch, embed = x.shape
    dim_codes, book_size, dim_embedding = codebook.shape
    cbt = codebook.transpose(0, 2, 1) * -2.0
    c_sq = jnp.sum(codebook * codebook, axis=2)[:, None, :]
    xt = x.reshape(batch, dim_codes, dim_embedding).transpose(1, 0, 2)

    grid = (batch // _B_BLK, dim_codes // _C_BLK)
    oh, idx3 = pl.pallas_call(
        _vq_tc_kernel,
        grid=grid,
        in_specs=[
            pl.BlockSpec((_C_BLK, _B_BLK, dim_embedding), lambda b, c: (c, b, 0)),
            pl.BlockSpec((_C_BLK, dim_embedding, book_size), lambda b, c: (c, 0, 0)),
            pl.BlockSpec((_C_BLK, 1, book_size), lambda b, c: (c, 0, 0)),
        ],
        out_specs=[
            pl.BlockSpec((_B_BLK, _C_BLK * book_size), lambda b, c: (b, c)),
            pl.BlockSpec((_C_BLK, _B_BLK, 1), lambda b, c: (c, b, 0)),
        ],
        out_shape=[
            jax.ShapeDtypeStruct((batch, dim_codes * book_size), jnp.float32),
            jax.ShapeDtypeStruct((dim_codes, batch, 1), jnp.int32),
        ],
    )(xt, cbt, c_sq)

    n_rows_total = dim_codes * batch
    n_workers = 32
    rows_per_w = n_rows_total // n_workers
    chunk = 128
    table = codebook.reshape(dim_codes * book_size, dim_embedding)
    idx_flat = idx3.reshape(n_rows_total // chunk, chunk)
    q = jnp.arange(n_rows_total, dtype=jnp.int32)
    dst = ((q % batch) * dim_codes + q // batch).reshape(
        n_rows_total // chunk, chunk)

    mesh = plsc.VectorSubcoreMesh(core_axis_name="c", subcore_axis_name="s")
    gather_call = pl.kernel(
        _vq_sc_gather,
        mesh=mesh,
        compiler_params=pltpu.CompilerParams(use_tc_tiling_on_sc=False),
        out_type=jax.ShapeDtypeStruct((n_rows_total, dim_embedding), jnp.float32),
        scratch_types=[
            pltpu.VMEM((rows_per_w // chunk, chunk), jnp.int32),
            pltpu.VMEM((rows_per_w // chunk, chunk), jnp.int32),
            pltpu.VMEM((rows_per_w, dim_embedding), jnp.float32),
            pltpu.SemaphoreType.DMA,
            pltpu.SemaphoreType.DMA,
        ],
    )
    ce_rows = gather_call(table, idx_flat, dst)

    ce = ce_rows.reshape(batch, embed)
    return (ce, ce, oh.reshape(batch, dim_codes, book_size))


# speedup vs baseline: 1.4135x; 1.3532x over previous
import jax
import jax.numpy as jnp
from jax import lax
from jax.experimental import pallas as pl
from jax.experimental.pallas import tpu as pltpu
from jax.experimental.pallas import tpu_sc as plsc

_B_BLK = 256
_C_BLK = 8


def _vq_tc_kernel(x_ref, cbt_ref, csq_ref, oh_ref, idx_ref):
    c_pid = pl.program_id(1)
    book_size = cbt_ref.shape[2]
    onehots = []
    for c in range(_C_BLK):
        xb = x_ref[c]
        cbt = cbt_ref[c]
        cross2 = jnp.dot(xb, cbt, preferred_element_type=jnp.float32)
        x_sq = jnp.sum(xb * xb, axis=1, keepdims=True)
        dist = (x_sq + cross2) + csq_ref[c]
        idx = jnp.argmin(dist, axis=1)
        k_iota = jax.lax.broadcasted_iota(jnp.int32, dist.shape, 1)
        onehot = (k_iota == idx[:, None]).astype(jnp.float32)
        oh_ref[:, c, :] = onehot
        c_glob = c_pid * _C_BLK + c
        idx_ref[c] = (idx + c_glob * book_size)[:, None]
    del onehots


def _vq_sc_gather(table_hbm, idx_hbm, dst_hbm, out_hbm, idx_v, dst_v, rows_v,
                  sem, sem2):
    wid = lax.axis_index("s") * 2 + lax.axis_index("c")
    n_chunks = idx_v.shape[0]
    chunk = idx_v.shape[1]
    pltpu.sync_copy(idx_hbm.at[pl.ds(wid * n_chunks, n_chunks)], idx_v)
    pltpu.sync_copy(dst_hbm.at[pl.ds(wid * n_chunks, n_chunks)], dst_v)
    gathers = []
    for j in range(n_chunks):
        gathers.append(pltpu.async_copy(
            table_hbm.at[idx_v.at[j]],
            rows_v.at[pl.ds(j * chunk, chunk)], sem))
    for cp in gathers:
        cp.wait()
    scatters = []
    for j in range(n_chunks):
        scatters.append(pltpu.async_copy(
            rows_v.at[pl.ds(j * chunk, chunk)],
            out_hbm.at[dst_v.at[j]], sem2))
    for cp in scatters:
        cp.wait()


def kernel(x, codebook):
    batch, embed = x.shape
    dim_codes, book_size, dim_embedding = codebook.shape
    cbt = codebook.transpose(0, 2, 1) * -2.0
    c_sq = jnp.sum(codebook * codebook, axis=2)[:, None, :]
    xt = x.reshape(batch, dim_codes, dim_embedding).transpose(1, 0, 2)

    grid = (batch // _B_BLK, dim_codes // _C_BLK)
    oh, idx3 = pl.pallas_call(
        _vq_tc_kernel,
        grid=grid,
        in_specs=[
            pl.BlockSpec((_C_BLK, _B_BLK, dim_embedding), lambda b, c: (c, b, 0)),
            pl.BlockSpec((_C_BLK, dim_embedding, book_size), lambda b, c: (c, 0, 0)),
            pl.BlockSpec((_C_BLK, 1, book_size), lambda b, c: (c, 0, 0)),
        ],
        out_specs=[
            pl.BlockSpec((_B_BLK, _C_BLK, book_size), lambda b, c: (b, c, 0)),
            pl.BlockSpec((_C_BLK, _B_BLK, 1), lambda b, c: (c, b, 0)),
        ],
        out_shape=[
            jax.ShapeDtypeStruct((batch, dim_codes, book_size), jnp.float32),
            jax.ShapeDtypeStruct((dim_codes, batch, 1), jnp.int32),
        ],
    )(xt, cbt, c_sq)

    n_rows_total = dim_codes * batch
    n_workers = 32
    rows_per_w = n_rows_total // n_workers
    chunk = 128
    table = codebook.reshape(dim_codes * book_size, dim_embedding)
    idx_flat = idx3.reshape(n_rows_total // chunk, chunk)
    q = jnp.arange(n_rows_total, dtype=jnp.int32)
    dst = ((q % batch) * dim_codes + q // batch).reshape(
        n_rows_total // chunk, chunk)

    mesh = plsc.VectorSubcoreMesh(core_axis_name="c", subcore_axis_name="s")
    gather_call = pl.kernel(
        _vq_sc_gather,
        mesh=mesh,
        compiler_params=pltpu.CompilerParams(use_tc_tiling_on_sc=False),
        out_type=jax.ShapeDtypeStruct((n_rows_total, dim_embedding), jnp.float32),
        scratch_types=[
            pltpu.VMEM((rows_per_w // chunk, chunk), jnp.int32),
            pltpu.VMEM((rows_per_w // chunk, chunk), jnp.int32),
            pltpu.VMEM((rows_per_w, dim_embedding), jnp.float32),
            pltpu.SemaphoreType.DMA,
            pltpu.SemaphoreType.DMA,
        ],
    )
    ce_rows = gather_call(table, idx_flat, dst)

    ce = ce_rows.reshape(batch, embed)
    return (ce, ce, oh)


# speedup vs baseline: 1.6498x; 1.1672x over previous
import jax
import jax.numpy as jnp
from jax import lax
from jax.experimental import pallas as pl
from jax.experimental.pallas import tpu as pltpu
from jax.experimental.pallas import tpu_sc as plsc

_B_BLK = 512
_C_BLK = 8


def _vq_tc_kernel(x_ref, cbt_ref, csq_ref, oh_ref, idx_ref):
    c_pid = pl.program_id(1)
    book_size = cbt_ref.shape[2]
    for c in range(_C_BLK):
        xb = x_ref[c]
        cbt = cbt_ref[c]
        cross2 = jnp.dot(xb, cbt, preferred_element_type=jnp.float32)
        x_sq = jnp.sum(xb * xb, axis=1, keepdims=True)
        dist = (x_sq + cross2) + csq_ref[c]
        idx = jnp.argmin(dist, axis=1)
        k_iota = jax.lax.broadcasted_iota(jnp.int32, dist.shape, 1)
        onehot = (k_iota == idx[:, None]).astype(jnp.float32)
        oh_ref[:, c, :] = onehot
        c_glob = c_pid * _C_BLK + c
        idx_ref[c] = (idx + c_glob * book_size)[:, None]


def _vq_sc_gather(table_hbm, idx_hbm, dst_hbm, out_hbm, idx_v, dst_v, rows_v,
                  sem, sem2):
    wid = lax.axis_index("s") * 2 + lax.axis_index("c")
    n_chunks = idx_v.shape[0]
    chunk = idx_v.shape[1]
    pltpu.sync_copy(idx_hbm.at[pl.ds(wid * n_chunks, n_chunks)], idx_v)
    pltpu.sync_copy(dst_hbm.at[pl.ds(wid * n_chunks, n_chunks)], dst_v)
    gathers = []
    for j in range(n_chunks):
        gathers.append(pltpu.async_copy(
            table_hbm.at[idx_v.at[j]],
            rows_v.at[pl.ds(j * chunk, chunk)], sem))
    for cp in gathers:
        cp.wait()
    scatters = []
    for j in range(n_chunks):
        scatters.append(pltpu.async_copy(
            rows_v.at[pl.ds(j * chunk, chunk)],
            out_hbm.at[dst_v.at[j]], sem2))
    for cp in scatters:
        cp.wait()


def kernel(x, codebook):
    batch, embed = x.shape
    dim_codes, book_size, dim_embedding = codebook.shape
    cbt = codebook.transpose(0, 2, 1) * -2.0
    c_sq = jnp.sum(codebook * codebook, axis=2)[:, None, :]
    xt = x.reshape(batch, dim_codes, dim_embedding).transpose(1, 0, 2)

    grid = (batch // _B_BLK, dim_codes // _C_BLK)
    oh, idx3 = pl.pallas_call(
        _vq_tc_kernel,
        grid=grid,
        in_specs=[
            pl.BlockSpec((_C_BLK, _B_BLK, dim_embedding), lambda b, c: (c, b, 0)),
            pl.BlockSpec((_C_BLK, dim_embedding, book_size), lambda b, c: (c, 0, 0)),
            pl.BlockSpec((_C_BLK, 1, book_size), lambda b, c: (c, 0, 0)),
        ],
        out_specs=[
            pl.BlockSpec((_B_BLK, _C_BLK, book_size), lambda b, c: (b, c, 0)),
            pl.BlockSpec((_C_BLK, _B_BLK, 1), lambda b, c: (c, b, 0)),
        ],
        out_shape=[
            jax.ShapeDtypeStruct((batch, dim_codes, book_size), jnp.float32),
            jax.ShapeDtypeStruct((dim_codes, batch, 1), jnp.int32),
        ],
    )(xt, cbt, c_sq)

    n_rows_total = dim_codes * batch
    n_workers = 32
    rows_per_w = n_rows_total // n_workers
    chunk = 128
    table = codebook.reshape(dim_codes * book_size, dim_embedding)
    idx_flat = idx3.reshape(n_rows_total // chunk, chunk)
    q = jnp.arange(n_rows_total, dtype=jnp.int32)
    dst = ((q % batch) * dim_codes + q // batch).reshape(
        n_rows_total // chunk, chunk)

    mesh = plsc.VectorSubcoreMesh(core_axis_name="c", subcore_axis_name="s")
    gather_call = pl.kernel(
        _vq_sc_gather,
        mesh=mesh,
        compiler_params=pltpu.CompilerParams(use_tc_tiling_on_sc=False),
        out_type=jax.ShapeDtypeStruct((n_rows_total, dim_embedding), jnp.float32),
        scratch_types=[
            pltpu.VMEM((rows_per_w // chunk, chunk), jnp.int32),
            pltpu.VMEM((rows_per_w // chunk, chunk), jnp.int32),
            pltpu.VMEM((rows_per_w, dim_embedding), jnp.float32),
            pltpu.SemaphoreType.DMA,
            pltpu.SemaphoreType.DMA,
        ],
    )
    ce_rows = gather_call(table, idx_flat, dst)

    ce = ce_rows.reshape(batch, embed)
    return (ce, ce, oh)
